# baseline (device time: 28693 ns/iter reference)
import jax
import jax.numpy as jnp
from jax import lax
from jax.experimental import pallas as pl
from jax.experimental.pallas import tpu as pltpu

N_DEV = 32
N_TOK = 1024
D_MODEL = 256
D_FF = 512
N_EXP = 128
E_LOCAL = N_EXP // N_DEV
ROWS = N_TOK // N_DEV
NZ = 4
NS = 8
PROW = N_TOK // NZ


def kernel(x, router_W, route_idx, expert_W):
    def body(x_ref, rw_ref, idx_ref, ew_ref, out_ref,
             partial_ref, zrecv_ref, s_ref, prec_ref, wle_ref,
             zs_sems, zr_sems, ps_sems, pr_sems):
        my = lax.axis_index("i")
        z = my // NS
        s = lax.rem(my, NS)

        bsem = pltpu.get_barrier_semaphore()
        for j in range(1, NZ):
            zmate = lax.rem(z + j, NZ) * NS + s
            pl.semaphore_signal(bsem, inc=1, device_id=(zmate,),
                                device_id_type=pl.DeviceIdType.MESH)
        for j in range(1, NS):
            pmate = z * NS + lax.rem(s + j, NS)
            pl.semaphore_signal(bsem, inc=1, device_id=(pmate,),
                                device_id_type=pl.DeviceIdType.MESH)
        pl.semaphore_wait(bsem, (NZ - 1) + (NS - 1))

        xf = x_ref[:, :]
        scores = jnp.dot(xf, rw_ref[:, :], preferred_element_type=jnp.float32)
        s_max = jnp.max(scores, axis=-1, keepdims=True)
        p = jnp.exp(scores - s_max)
        probs = p / jnp.sum(p, axis=-1, keepdims=True)

        e0 = idx_ref[:, 0:1]
        e1 = idx_ref[:, 1:2]
        eids = lax.broadcasted_iota(jnp.int32, (N_TOK, N_EXP), 1)
        g0 = jnp.sum(jnp.where(eids == e0, probs, 0.0), axis=1, keepdims=True)
        g1 = jnp.sum(jnp.where(eids == e1, probs, 0.0), axis=1, keepdims=True)
        gs = g0 + g1
        w0 = g0 / gs
        w1 = g1 / gs

        wles = []
        for le in range(E_LOCAL):
            ge = my * E_LOCAL + le
            wle = jnp.where(e0 == ge, w0, 0.0) + jnp.where(e1 == ge, w1, 0.0)
            wles.append(wle.astype(jnp.bfloat16))
        wle_ref[...] = jnp.concatenate(wles, axis=1)

        wcat = ew_ref[...].reshape(E_LOCAL * D_MODEL, D_FF).astype(jnp.bfloat16)

        def compute_block(zp):
            r0 = zp * PROW
            xg = x_ref[pl.ds(r0, PROW), :].astype(jnp.bfloat16)
            wg = wle_ref[pl.ds(r0, PROW), :]
            xcat = jnp.concatenate(
                [xg * wg[:, le:le + 1] for le in range(E_LOCAL)], axis=1)
            acc = jnp.dot(xcat, wcat, preferred_element_type=jnp.float32)
            partial_ref[pl.ds(r0, PROW), :] = acc.astype(jnp.bfloat16)

        zsends = []
        for j in range(1, NZ):
            zp = lax.rem(z + j, NZ)
            compute_block(zp)
            rdma = pltpu.make_async_remote_copy(
                src_ref=partial_ref.at[pl.ds(zp * PROW, PROW), :],
                dst_ref=zrecv_ref.at[j - 1],
                send_sem=zs_sems.at[j - 1],
                recv_sem=zr_sems.at[j - 1],
                device_id=(zp * NS + s,),
                device_id_type=pl.DeviceIdType.MESH,
            )
            rdma.start()
            zsends.append(rdma)
        compute_block(z)

        for rdma in zsends:
            rdma.wait_recv()

        ssum = partial_ref[pl.ds(z * PROW, PROW), :].astype(jnp.float32)
        ssum = ssum + jnp.sum(zrecv_ref[...].astype(jnp.float32), axis=0)
        s_ref[...] = ssum.astype(jnp.bfloat16)

        psends = []
        for j in range(1, NS):
            t = lax.rem(s + j, NS)
            rdma = pltpu.make_async_remote_copy(
                src_ref=s_ref.at[pl.ds(t * ROWS, ROWS), :],
                dst_ref=prec_ref.at[j - 1],
                send_sem=ps_sems.at[j - 1],
                recv_sem=pr_sems.at[j - 1],
                device_id=(z * NS + t,),
                device_id_type=pl.DeviceIdType.MESH,
            )
            rdma.start()
            psends.append(rdma)

        for rdma in psends:
            rdma.wait_recv()

        own = s_ref[pl.ds(s * ROWS, ROWS), :].astype(jnp.float32)
        out_ref[:, :] = own + jnp.sum(prec_ref[...].astype(jnp.float32), axis=0)

        for rdma in zsends:
            rdma.wait_send()
        for rdma in psends:
            rdma.wait_send()

    return pl.pallas_call(
        body,
        out_shape=jax.ShapeDtypeStruct((ROWS, D_FF), jnp.float32),
        in_specs=[
            pl.BlockSpec(memory_space=pltpu.VMEM),
            pl.BlockSpec(memory_space=pltpu.VMEM),
            pl.BlockSpec(memory_space=pltpu.VMEM),
            pl.BlockSpec(memory_space=pltpu.VMEM),
        ],
        out_specs=pl.BlockSpec(memory_space=pltpu.VMEM),
        scratch_shapes=[
            pltpu.VMEM((N_TOK, D_FF), jnp.bfloat16),
            pltpu.VMEM((NZ - 1, PROW, D_FF), jnp.bfloat16),
            pltpu.VMEM((PROW, D_FF), jnp.bfloat16),
            pltpu.VMEM((NS - 1, ROWS, D_FF), jnp.bfloat16),
            pltpu.VMEM((N_TOK, E_LOCAL), jnp.bfloat16),
            pltpu.SemaphoreType.DMA((NZ - 1,)),
            pltpu.SemaphoreType.DMA((NZ - 1,)),
            pltpu.SemaphoreType.DMA((NS - 1,)),
            pltpu.SemaphoreType.DMA((NS - 1,)),
        ],
        compiler_params=pltpu.CompilerParams(collective_id=0),
    )(x, router_W, route_idx, expert_W)


# device time: 24035 ns/iter; 1.1938x vs baseline; 1.1938x over previous
import jax
import jax.numpy as jnp
from jax import lax
from jax.experimental import pallas as pl
from jax.experimental.pallas import tpu as pltpu

N_DEV = 32
N_TOK = 1024
D_MODEL = 256
D_FF = 512
N_EXP = 128
E_LOCAL = N_EXP // N_DEV
ROWS = N_TOK // N_DEV
NZ = 4
NS = 8
PROW = N_TOK // NZ
CAP = 64


def kernel(x, router_W, route_idx, expert_W):
    def body(x_ref, rw_ref, idx_ref, ew_ref, out_ref,
             own_ref, zsend_ref, zrecv_ref, s_ref, prec_ref,
             wle_ref, m_ref,
             zs_sems, zr_sems, ps_sems, pr_sems):
        my = lax.axis_index("i")
        z = my // NS
        s = lax.rem(my, NS)

        bsem = pltpu.get_barrier_semaphore()
        for j in range(1, NZ):
            zmate = lax.rem(z + j, NZ) * NS + s
            pl.semaphore_signal(bsem, inc=1, device_id=(zmate,),
                                device_id_type=pl.DeviceIdType.MESH)
        for j in range(1, NS):
            pmate = z * NS + lax.rem(s + j, NS)
            pl.semaphore_signal(bsem, inc=1, device_id=(pmate,),
                                device_id_type=pl.DeviceIdType.MESH)
        pl.semaphore_wait(bsem, (NZ - 1) + (NS - 1))

        xf = x_ref[:, :]
        scores = jnp.dot(xf, rw_ref[:, :], preferred_element_type=jnp.float32)
        s_max = jnp.max(scores, axis=-1, keepdims=True)
        p = jnp.exp(scores - s_max)
        probs = p / jnp.sum(p, axis=-1, keepdims=True)

        e0 = idx_ref[:, 0:1]
        e1 = idx_ref[:, 1:2]
        eids = lax.broadcasted_iota(jnp.int32, (N_TOK, N_EXP), 1)
        g0 = jnp.sum(jnp.where(eids == e0, probs, 0.0), axis=1, keepdims=True)
        g1 = jnp.sum(jnp.where(eids == e1, probs, 0.0), axis=1, keepdims=True)
        gs = g0 + g1
        w0 = g0 / gs
        w1 = g1 / gs

        wles = []
        for le in range(E_LOCAL):
            ge = my * E_LOCAL + le
            wle = jnp.where(e0 == ge, w0, 0.0) + jnp.where(e1 == ge, w1, 0.0)
            wles.append(wle.astype(jnp.bfloat16))
        wle_ref[...] = jnp.concatenate(wles, axis=1)

        c0 = e0 // E_LOCAL
        c1 = e1 // E_LOCAL
        mcols = []
        for k in range(NZ):
            dk = lax.rem(z + k, NZ) * NS + s
            mk = jnp.where((c0 == dk) | (c1 == dk), 1.0, 0.0)
            mcols.append(mk)
        m_ref[...] = jnp.concatenate(mcols, axis=1)

        ri = lax.broadcasted_iota(jnp.int32, (PROW, PROW), 0)
        ci = lax.broadcasted_iota(jnp.int32, (PROW, PROW), 1)
        tril = jnp.where(ci < ri, 1.0, 0.0)

        wcat = ew_ref[...].reshape(E_LOCAL * D_MODEL, D_FF).astype(jnp.bfloat16)

        def block_partial(zp):
            r0 = zp * PROW
            xg = x_ref[pl.ds(r0, PROW), :].astype(jnp.bfloat16)
            wg = wle_ref[pl.ds(r0, PROW), :]
            xcat = jnp.concatenate(
                [xg * wg[:, le:le + 1] for le in range(E_LOCAL)], axis=1)
            acc = jnp.dot(xcat, wcat, preferred_element_type=jnp.float32)
            mb = m_ref[pl.ds(r0, PROW), :]
            ranks = jnp.dot(tril, mb, preferred_element_type=jnp.float32)
            return acc, mb, ranks

        cap_i = lax.broadcasted_iota(jnp.int32, (CAP, PROW), 0)
        zsends = []
        for j in range(1, NZ):
            zp = lax.rem(z + j, NZ)
            acc, mb, ranks = block_partial(zp)
            rk = jnp.transpose(ranks[:, 0:1])
            mbr = jnp.transpose(mb[:, 0:1])
            oh = jnp.where((cap_i == rk.astype(jnp.int32)) & (mbr > 0.5),
                           1.0, 0.0).astype(jnp.bfloat16)
            cblk = jnp.dot(oh, acc.astype(jnp.bfloat16),
                           preferred_element_type=jnp.float32)
            zsend_ref[j - 1] = cblk.astype(jnp.bfloat16)
            rdma = pltpu.make_async_remote_copy(
                src_ref=zsend_ref.at[j - 1],
                dst_ref=zrecv_ref.at[j - 1],
                send_sem=zs_sems.at[j - 1],
                recv_sem=zr_sems.at[j - 1],
                device_id=(zp * NS + s,),
                device_id_type=pl.DeviceIdType.MESH,
            )
            rdma.start()
            zsends.append(rdma)

        acc_own, mb_own, ranks_own = block_partial(z)
        own_ref[...] = acc_own

        for rdma in zsends:
            rdma.wait_recv()

        row_i = lax.broadcasted_iota(jnp.int32, (PROW, CAP), 1)
        ssum = own_ref[...]
        for j in range(1, NZ):
            col = NZ - j
            rk = ranks_own[:, col:col + 1]
            mbr = mb_own[:, col:col + 1]
            oht = jnp.where((row_i == rk.astype(jnp.int32)) & (mbr > 0.5),
                            1.0, 0.0).astype(jnp.bfloat16)
            ssum = ssum + jnp.dot(oht, zrecv_ref[j - 1],
                                  preferred_element_type=jnp.float32)
        s_ref[...] = ssum.astype(jnp.bfloat16)

        psends = []
        for j in range(1, NS):
            t = lax.rem(s + j, NS)
            rdma = pltpu.make_async_remote_copy(
                src_ref=s_ref.at[pl.ds(t * ROWS, ROWS), :],
                dst_ref=prec_ref.at[j - 1],
                send_sem=ps_sems.at[j - 1],
                recv_sem=pr_sems.at[j - 1],
                device_id=(z * NS + t,),
                device_id_type=pl.DeviceIdType.MESH,
            )
            rdma.start()
            psends.append(rdma)

        for rdma in psends:
            rdma.wait_recv()

        own = s_ref[pl.ds(s * ROWS, ROWS), :].astype(jnp.float32)
        out_ref[:, :] = own + jnp.sum(prec_ref[...].astype(jnp.float32), axis=0)

        for rdma in zsends:
            rdma.wait_send()
        for rdma in psends:
            rdma.wait_send()

    return pl.pallas_call(
        body,
        out_shape=jax.ShapeDtypeStruct((ROWS, D_FF), jnp.float32),
        in_specs=[
            pl.BlockSpec(memory_space=pltpu.VMEM),
            pl.BlockSpec(memory_space=pltpu.VMEM),
            pl.BlockSpec(memory_space=pltpu.VMEM),
            pl.BlockSpec(memory_space=pltpu.VMEM),
        ],
        out_specs=pl.BlockSpec(memory_space=pltpu.VMEM),
        scratch_shapes=[
            pltpu.VMEM((PROW, D_FF), jnp.float32),
            pltpu.VMEM((NZ - 1, CAP, D_FF), jnp.bfloat16),
            pltpu.VMEM((NZ - 1, CAP, D_FF), jnp.bfloat16),
            pltpu.VMEM((PROW, D_FF), jnp.bfloat16),
            pltpu.VMEM((NS - 1, ROWS, D_FF), jnp.bfloat16),
            pltpu.VMEM((N_TOK, E_LOCAL), jnp.bfloat16),
            pltpu.VMEM((N_TOK, NZ), jnp.float32),
            pltpu.SemaphoreType.DMA((NZ - 1,)),
            pltpu.SemaphoreType.DMA((NZ - 1,)),
            pltpu.SemaphoreType.DMA((NS - 1,)),
            pltpu.SemaphoreType.DMA((NS - 1,)),
        ],
        compiler_params=pltpu.CompilerParams(collective_id=0),
    )(x, router_W, route_idx, expert_W)
